# baseline (speedup 1.0000x reference)
import jax
import jax.numpy as jnp
from jax import lax
from jax.experimental import pallas as pl
from jax.experimental.pallas import tpu as pltpu

_ELECTRODE_POS = (
    (1, 0), (2, 8), (3, 9), (4, 17), (5, 18), (6, 26),
    (7, 21), (8, 23), (9, 30), (10, 31), (11, 32), (12, 39),
    (13, 40), (14, 41), (15, 48), (16, 49), (17, 50),
)

_ROWS = 16384
_W = 3456
_BLOCK_ROWS = 256


def _body(removed_ref, x_ref, o_ref):
    col = lax.broadcasted_iota(jnp.int32, (1, _W), 1) % 54
    zero = jnp.zeros((1, _W), dtype=jnp.bool_)
    n_removed = removed_ref.shape[1]
    for e, p in _ELECTRODE_POS:
        present = removed_ref[0, 0] == e
        for j in range(1, n_removed):
            present = jnp.logical_or(present, removed_ref[0, j] == e)
        zero = jnp.logical_or(zero, jnp.logical_and(present, col == p))
    o_ref[...] = jnp.where(zero, jnp.float32(0.0), x_ref[...])


def kernel(x, removed_electrodes):
    xf = x.reshape(_ROWS, _W)
    rem = removed_electrodes.astype(jnp.int32).reshape(1, -1)
    out = pl.pallas_call(
        _body,
        grid=(_ROWS // _BLOCK_ROWS,),
        in_specs=[
            pl.BlockSpec(memory_space=pltpu.SMEM),
            pl.BlockSpec((_BLOCK_ROWS, _W), lambda i: (i, 0)),
        ],
        out_specs=pl.BlockSpec((_BLOCK_ROWS, _W), lambda i: (i, 0)),
        out_shape=jax.ShapeDtypeStruct((_ROWS, _W), jnp.float32),
    )(rem, xf)
    return out.reshape(x.shape)


# speedup vs baseline: 43.5362x; 43.5362x over previous
import jax
import jax.numpy as jnp
from jax import lax
from jax.experimental import pallas as pl
from jax.experimental.pallas import tpu as pltpu

_ELECTRODE_POS = (
    (1, 0), (2, 8), (3, 9), (4, 17), (5, 18), (6, 26),
    (7, 21), (8, 23), (9, 30), (10, 31), (11, 32), (12, 39),
    (13, 40), (14, 41), (15, 48), (16, 49), (17, 50),
)

_NPOS = 54
_BATCH = 64
_ROWS = _NPOS * _BATCH
_H, _W = 32, 512
_BLOCK_ROWS = 54


def _body(removed_ref, x_ref, o_ref):
    pos = lax.broadcasted_iota(jnp.int32, (_BLOCK_ROWS, 1, 1), 0)
    zero = jnp.zeros((_BLOCK_ROWS, 1, 1), dtype=jnp.bool_)
    n_removed = removed_ref.shape[1]
    for e, p in _ELECTRODE_POS:
        present = removed_ref[0, 0] == e
        for j in range(1, n_removed):
            present = jnp.logical_or(present, removed_ref[0, j] == e)
        zero = jnp.logical_or(zero, jnp.logical_and(present, pos == p))
    o_ref[...] = jnp.where(zero, jnp.float32(0.0), x_ref[...])


def kernel(x, removed_electrodes):
    xt = jnp.transpose(x, (0, 3, 4, 1, 2)).reshape(_ROWS, _H, _W)
    rem = removed_electrodes.astype(jnp.int32).reshape(1, -1)
    out = pl.pallas_call(
        _body,
        grid=(_ROWS // _BLOCK_ROWS,),
        in_specs=[
            pl.BlockSpec(memory_space=pltpu.SMEM),
            pl.BlockSpec((_BLOCK_ROWS, _H, _W), lambda i: (i, 0, 0)),
        ],
        out_specs=pl.BlockSpec((_BLOCK_ROWS, _H, _W), lambda i: (i, 0, 0)),
        out_shape=jax.ShapeDtypeStruct((_ROWS, _H, _W), jnp.float32),
    )(rem, xt)
    return jnp.transpose(out.reshape(_BATCH, 6, 9, _H, _W), (0, 3, 4, 1, 2))
